# baseline (speedup 1.0000x reference)
import jax
import jax.numpy as jnp
from jax.experimental import pallas as pl

_N_USERS = 50000
_N_ITEMS = 50000
_N_NODES = _N_USERS + _N_ITEMS
_N_LAYERS = 3


def _finish_body(au_ref, ai_ref, u_ref, i_ref):
    u_ref[...] = au_ref[...] * 0.25
    i_ref[...] = ai_ref[...] * 0.25


def kernel(adj_index, adj_values, user_emb, user_emb_pre, item_emb, item_emb_pre):
    ego = jnp.concatenate([user_emb + user_emb_pre, item_emb + item_emb_pre], axis=0)
    acc = ego
    row = adj_index[0]
    col = adj_index[1]
    for _ in range(_N_LAYERS):
        msg = jnp.take(ego, col, axis=0) * adj_values[:, None]
        ego = jax.ops.segment_sum(msg, row, num_segments=_N_NODES)
        acc = acc + ego
    au = acc[:_N_USERS].reshape(_N_USERS * 32 // 128, 128)
    ai = acc[_N_USERS:].reshape(_N_ITEMS * 32 // 128, 128)
    nrows = au.shape[0]
    u, i = pl.pallas_call(
        _finish_body,
        out_shape=[
            jax.ShapeDtypeStruct((nrows, 128), jnp.float32),
            jax.ShapeDtypeStruct((nrows, 128), jnp.float32),
        ],
    )(au, ai)
    return (u.reshape(_N_USERS, 32), i.reshape(_N_ITEMS, 32))


# speedup vs baseline: 10.9548x; 10.9548x over previous
import jax
import jax.numpy as jnp
from jax import lax
from jax.experimental import pallas as pl
from jax.experimental.pallas import tpu as pltpu
from jax.experimental.pallas import tpu_sc as plsc

_N_USERS = 50000
_N_ITEMS = 50000
_N = _N_USERS + _N_ITEMS
_E = 1600000
_NS = 16

_NP = 100096
_RPT = _NP // _NS
_ZB = 184

_CHUNK = 1024
_G = 128
_GP = _CHUNK // _G
_CR = _CHUNK // 128
_EPT = 100352
_NCHUNK = _EPT // _CHUNK
_EPAD = _EPT * _NS
_ERB = _EPAD // 128

_BR = 2000


def _colcat_body(col_ref, lo_ref, hi_ref):
    lo_ref[...] = col_ref[...]
    hi_ref[...] = col_ref[...] + _NP


def _pro_body(a_ref, b_ref, o_ref):
    s = a_ref[...] + b_ref[...]
    o_ref[0] = s[:, :16]
    o_ref[1] = s[:, 16:]


def _epi_body(e0_ref, e1_ref, e2_ref, e3_ref, o_ref):
    for c in range(2):
        s = (e0_ref[c] + e1_ref[c]) + (e2_ref[c] + e3_ref[c])
        o_ref[:, 16 * c:16 * (c + 1)] = s * 0.25


def _sc_body(colcat, rowr, valr, e0,
             e1, e2, e3,
             acc, gsem):
    f32 = jnp.float32
    i32 = jnp.int32
    pl.run_scoped(
        lambda colbuf, rowbuf, valbuf, rows, zbuf, wtmp: _sc_inner(
            colcat, rowr, valr, e0, e1, e2, e3, acc, gsem,
            colbuf, rowbuf, valbuf, rows, zbuf, wtmp),
        pltpu.VMEM((_CR, 128), i32),
        pltpu.VMEM((_CR, 128), i32),
        pltpu.VMEM((_CR, 128), f32),
        pltpu.VMEM((_CHUNK, 16), f32),
        pltpu.VMEM((_ZB, 16), f32),
        pltpu.VMEM((_ZB, 16), f32),
    )


def _sc_inner(colcat, rowr, valr, e0, e1, e2, e3, acc, gsem,
              colbuf, rowbuf, valbuf, rows, zbuf, wtmp):
    c = lax.axis_index("c")
    s = lax.axis_index("s")
    half = c * _NP

    zvec = jnp.zeros((16,), jnp.float32)

    def zb_body(i, _):
        zbuf[i] = zvec
        return 0

    lax.fori_loop(0, _ZB, zb_body, 0)
    for k in range(_RPT // _ZB):
        pltpu.sync_copy(zbuf, acc.at[pl.ds(s * _RPT + k * _ZB, _ZB)])
    plsc.subcore_barrier()

    ebase = s * (_EPT // 128)
    cbase = c * _ERB + ebase
    for src, dst in ((e0, e1), (e1, e2), (e2, e3)):

        def chunk_body(ch, _, src=src):
            r0 = ebase + ch * _CR
            pltpu.sync_copy(colcat.at[pl.ds(cbase + ch * _CR, _CR)], colbuf)
            pltpu.sync_copy(rowr.at[pl.ds(r0, _CR)], rowbuf)
            pltpu.sync_copy(valr.at[pl.ds(r0, _CR)], valbuf)
            cps = [pltpu.async_copy(src.at[colbuf.at[j]],
                                    rows.at[pl.ds(j * _G, _G)], gsem)
                   for j in range(_GP)]
            for cp in cps:
                cp.wait()

            def scale_body(g, _):
                jj = g // 8
                tt = g - jj * 8
                vv = valbuf[jj, pl.ds(tt * 16, 16)]
                base = g * 16
                for e in range(16):
                    sv = lax.broadcast(vv[e], (16,))
                    rows[base + e] = rows[base + e] * sv
                return 0

            lax.fori_loop(0, _CHUNK // 16, scale_body, 0)
            for j in range(_GP):
                pltpu.sync_copy(rows.at[pl.ds(j * _G, _G)],
                                acc.at[rowbuf.at[j]], add=True)
            return 0

        lax.fori_loop(0, _NCHUNK, chunk_body, 0)
        plsc.subcore_barrier()

        def wb_body(k, _, dst=dst):
            pltpu.sync_copy(acc.at[pl.ds(s * _RPT + k * _ZB, _ZB)], wtmp)
            pltpu.sync_copy(wtmp, dst.at[pl.ds(half + s * _RPT + k * _ZB, _ZB)])
            return 0

        lax.fori_loop(0, _RPT // _ZB, wb_body, 0)
        for k in range(_RPT // _ZB):
            pltpu.sync_copy(zbuf, acc.at[pl.ds(s * _RPT + k * _ZB, _ZB)])
        plsc.subcore_barrier()


def kernel(adj_index, adj_values, user_emb, user_emb_pre, item_emb, item_emb_pre):
    f32 = jnp.float32
    i32 = jnp.int32
    pad = _EPAD - _E
    col = jnp.concatenate([adj_index[1], jnp.zeros((pad,), i32)])
    row = jnp.concatenate([adj_index[0], jnp.zeros((pad,), i32)])
    val = jnp.concatenate([adj_values, jnp.zeros((pad,), f32)])
    colr = col.reshape(_ERB, 128)
    rowr = row.reshape(_ERB, 128)
    valr = val.reshape(_ERB, 128)

    lo, hi = pl.pallas_call(
        _colcat_body,
        out_shape=[
            jax.ShapeDtypeStruct((_ERB, 128), i32),
            jax.ShapeDtypeStruct((_ERB, 128), i32),
        ],
    )(colr)
    colcat = jnp.concatenate([lo, hi], axis=0)

    allemb = jnp.concatenate([user_emb, item_emb], axis=0)
    allpre = jnp.concatenate([user_emb_pre, item_emb_pre], axis=0)
    e0 = pl.pallas_call(
        _pro_body,
        grid=(_N // _BR,),
        in_specs=[
            pl.BlockSpec((_BR, 32), lambda r: (r, 0)),
            pl.BlockSpec((_BR, 32), lambda r: (r, 0)),
        ],
        out_specs=pl.BlockSpec((2, _BR, 16), lambda r: (0, r, 0)),
        out_shape=jax.ShapeDtypeStruct((2, _NP, 16), f32),
    )(allemb, allpre)

    mesh = plsc.VectorSubcoreMesh(core_axis_name="c", subcore_axis_name="s")
    e1, e2, e3 = pl.kernel(
        _sc_body,
        out_type=[
            jax.ShapeDtypeStruct((2 * _NP, 16), f32),
            jax.ShapeDtypeStruct((2 * _NP, 16), f32),
            jax.ShapeDtypeStruct((2 * _NP, 16), f32),
        ],
        mesh=mesh,
        compiler_params=pltpu.CompilerParams(use_tc_tiling_on_sc=False),
        scratch_types=[
            pltpu.VMEM_SHARED((_NP, 16), f32),
            pltpu.SemaphoreType.DMA,
        ],
    )(colcat, rowr, valr, e0.reshape(2 * _NP, 16))

    es = [e0.reshape(2, _NP, 16)] + [e.reshape(2, _NP, 16) for e in (e1, e2, e3)]
    outs = []
    for part in range(2):
        off = part * (_N_USERS // _BR)
        o = pl.pallas_call(
            _epi_body,
            grid=(_N_USERS // _BR,),
            in_specs=[pl.BlockSpec((2, _BR, 16), lambda r, off=off: (0, off + r, 0))
                      for _ in range(4)],
            out_specs=pl.BlockSpec((_BR, 32), lambda r: (r, 0)),
            out_shape=jax.ShapeDtypeStruct((_N_USERS, 32), f32),
        )(*es)
        outs.append(o)
    return (outs[0], outs[1])
